# baseline (speedup 1.0000x reference)
import functools

import jax
import jax.numpy as jnp
from jax import lax
from jax.experimental import pallas as pl
from jax.experimental.pallas import tpu as pltpu
from jax.experimental.pallas import tpu_sc as plsc

VOCAB = 100000
EMB = 100
EMBP = 128
NVREG = EMBP // 16
B = 16384
NC, NS = 2, 16
NW = NC * NS
BPW = B // NW
T_LEN, D_LEN = 20, 200
DG = 16
TG = 8
H1, H2 = 512, 256
BM = 512
F32 = jnp.float32


def _sc_bag_sums(table_p, tflat, dflat, yflat, nb):
  mesh = plsc.VectorSubcoreMesh(core_axis_name="c", subcore_axis_name="s")
  bpw = nb // NW

  @functools.partial(
      pl.kernel,
      out_type=(
          jax.ShapeDtypeStruct((nb, EMBP), F32),
          jax.ShapeDtypeStruct((nb, EMBP), F32),
          jax.ShapeDtypeStruct((nb, EMBP), F32),
      ),
      mesh=mesh,
      scratch_types=[
          pltpu.VMEM((D_LEN, EMBP), F32),
          pltpu.VMEM((D_LEN, EMBP), F32),
          pltpu.VMEM((D_LEN, EMBP), F32),
          pltpu.VMEM((D_LEN, EMBP), F32),
          pltpu.VMEM((bpw * T_LEN,), jnp.int32),
          pltpu.VMEM((DG * D_LEN,), jnp.int32),
          pltpu.VMEM((DG * D_LEN,), jnp.int32),
          pltpu.VMEM((bpw,), jnp.int32),
          pltpu.VMEM((DG, EMBP), F32),
          pltpu.SemaphoreType.DMA,
          pltpu.SemaphoreType.DMA,
          pltpu.SemaphoreType.DMA,
          pltpu.SemaphoreType.DMA,
          pltpu.SemaphoreType.DMA,
          pltpu.SemaphoreType.DMA,
      ],
  )
  def body(table, tidx_h, didx_h, yidx_h, t_out, d_out, y_out,
           buf0, buf1, buf2, buf3, tidx_v, didx0, didx1, yidx_v, stage,
           semA, semB, semC, semD, semI0, semI1):
    wid = lax.axis_index("s") * NC + lax.axis_index("c")
    bag0 = wid * bpw
    bufs = (buf0, buf1, buf2, buf3)
    sems = (semA, semB, semC, semD)
    didxs = (didx0, didx1)
    semis = (semI0, semI1)

    zeros = tuple(jnp.zeros((16,), F32) for _ in range(NVREG))

    def accum_rows(buf, row0, nrows, unroll):
      def row_add(j, accs):
        r = row0 + unroll * j
        for u in range(unroll):
          accs = tuple(accs[k] + buf[r + u, pl.ds(16 * k, 16)]
                       for k in range(NVREG))
        return accs
      return lax.fori_loop(0, nrows // unroll, row_add, zeros)

    def store_stage(i, accs):
      for k in range(NVREG):
        stage[i, pl.ds(16 * k, 16)] = accs[k]

    def issue_rows(idx_ref, off, n0, n1, buf, sem):
      pltpu.async_copy(table.at[idx_ref.at[pl.ds(off, n0)]],
                       buf.at[pl.ds(0, n0)], sem)
      pltpu.async_copy(table.at[idx_ref.at[pl.ds(off + n0, n1)]],
                       buf.at[pl.ds(n0, n1)], sem)

    def wait_rows(n, buf, sem):
      pltpu.make_async_copy(table.at[pl.ds(0, n)],
                            buf.at[pl.ds(0, n)], sem).wait()

    NYC = bpw // 128
    pltpu.sync_copy(yidx_h.at[pl.ds(bag0, bpw)], yidx_v)
    pltpu.async_copy(table.at[yidx_v.at[pl.ds(0, 128)]],
                     bufs[0].at[pl.ds(0, 128)], sems[0])
    for c in range(NYC):
      if c + 1 < NYC:
        off = pl.multiple_of((c + 1) * 128, 8)
        pltpu.async_copy(table.at[yidx_v.at[pl.ds(off, 128)]],
                         bufs[(c + 1) % 2].at[pl.ds(0, 128)],
                         sems[(c + 1) % 2])
      wait_rows(128, bufs[c % 2], sems[c % 2])
      pltpu.sync_copy(bufs[c % 2].at[pl.ds(0, 128)],
                      y_out.at[pl.ds(bag0 + c * 128, 128)])

    pltpu.sync_copy(tidx_h.at[pl.ds(bag0 * T_LEN, bpw * T_LEN)], tidx_v)
    NTG = bpw // TG
    def issue_tgroup(g, p):
      off = pl.multiple_of(jnp.minimum(g, NTG - 1) * (TG * T_LEN), 8)
      issue_rows(tidx_v, off, 104, 56, bufs[p], sems[p])
    for p in range(3):
      issue_tgroup(p, p)
    def title_quad(sb, _):
      for gp in range(4):
        g = 4 * sb + gp
        issue_tgroup(g + 3, (gp + 3) % 4)
        wait_rows(160, bufs[gp], sems[gp])
        def title_bag(i, _):
          accs = accum_rows(bufs[gp], i * T_LEN, T_LEN, 4)
          store_stage(i, accs)
          return 0
        lax.fori_loop(0, TG, title_bag, 0)
        pltpu.sync_copy(stage.at[pl.ds(0, TG)],
                        t_out.at[pl.ds(bag0 + g * TG, TG)])
      return 0
    lax.fori_loop(0, NTG // 4, title_quad, 0)
    for p in range(3):
      wait_rows(160, bufs[p], sems[p])

    NDG = bpw // DG
    def issue_didx(g, gp):
      goff = bag0 * D_LEN + jnp.minimum(g, NDG - 1) * (DG * D_LEN)
      pltpu.async_copy(didx_h.at[pl.ds(pl.multiple_of(goff, 8), DG * D_LEN)],
                       didxs[gp], semis[gp])
    def wait_didx(gp):
      pltpu.make_async_copy(didx_h.at[pl.ds(0, DG * D_LEN)],
                            didxs[gp], semis[gp]).wait()

    def issue_bag(idx_ref, i, p):
      off = pl.multiple_of(i * D_LEN, 8)
      issue_rows(idx_ref, off, 104, 96, bufs[p], sems[p])

    def do_bag(i, p):
      wait_rows(D_LEN, bufs[p], sems[p])
      store_stage(i, accum_rows(bufs[p], 0, D_LEN, 4))

    issue_didx(0, 0)
    wait_didx(0)
    issue_didx(1, 1)
    for p in range(3):
      issue_bag(didxs[0], p, p)

    def desc_pair(sb, _):
      for gp in range(2):
        g = 2 * sb + gp
        cidx, nidx = didxs[gp], didxs[1 - gp]
        def desc_quad(q, _):
          for p in range(4):
            n = 4 * q + p
            issue_bag(cidx, n + 3, (p + 3) % 4)
            do_bag(n, p)
          return 0
        lax.fori_loop(0, 3, desc_quad, 0)
        issue_bag(cidx, 15, 3)
        do_bag(12, 0)
        wait_didx(1 - gp)
        issue_bag(nidx, 0, 0)
        do_bag(13, 1)
        issue_bag(nidx, 1, 1)
        do_bag(14, 2)
        issue_bag(nidx, 2, 2)
        do_bag(15, 3)
        issue_didx(g + 2, gp)
        pltpu.sync_copy(stage, d_out.at[pl.ds(bag0 + g * DG, DG)])
      return 0
    lax.fori_loop(0, NDG // 2, desc_pair, 0)
    for p in range(3):
      wait_rows(D_LEN, bufs[p], sems[p])
    wait_didx(1)

  return body(table_p, tflat, dflat, yflat)


def _pad_body(x_ref, o_ref):
  o_ref[...] = jnp.concatenate(
      [x_ref[...], jnp.zeros((x_ref.shape[0], EMBP - EMB), F32)], axis=1)


def _pad_table(emb):
  rb = 1000
  return pl.pallas_call(
      _pad_body,
      grid=(VOCAB // rb,),
      in_specs=[pl.BlockSpec((rb, EMB), lambda i: (i, 0))],
      out_specs=pl.BlockSpec((rb, EMBP), lambda i: (i, 0)),
      out_shape=jax.ShapeDtypeStruct((VOCAB, EMBP), F32),
  )(emb)


def _mlp_body(t_ref, d_ref, y_ref, w1a, w1b, w1c, b1, w2, b2, w3, out_ref):
  bf = jnp.bfloat16
  h = (jnp.dot(t_ref[...].astype(bf), w1a[...], preferred_element_type=F32)
       + jnp.dot(d_ref[...].astype(bf), w1b[...], preferred_element_type=F32)
       + jnp.dot(y_ref[...].astype(bf), w1c[...], preferred_element_type=F32)
       + b1[...])
  h = jnp.maximum(h, 0.0)
  h2 = jnp.maximum(
      jnp.dot(h.astype(bf), w2[...], preferred_element_type=F32) + b2[...],
      0.0)
  out_ref[...] = jnp.dot(h2, w3[...], preferred_element_type=F32)


def _mlp(t, d, y, w1a, w1b, w1c, b1_2d, w2, b2_2d, w3_2d):
  grid = t.shape[0] // BM
  return pl.pallas_call(
      _mlp_body,
      grid=(grid,),
      in_specs=[
          pl.BlockSpec((BM, EMBP), lambda i: (i, 0)),
          pl.BlockSpec((BM, EMBP), lambda i: (i, 0)),
          pl.BlockSpec((BM, EMBP), lambda i: (i, 0)),
          pl.BlockSpec((EMBP, H1), lambda i: (0, 0)),
          pl.BlockSpec((EMBP, H1), lambda i: (0, 0)),
          pl.BlockSpec((EMBP, H1), lambda i: (0, 0)),
          pl.BlockSpec((1, H1), lambda i: (0, 0)),
          pl.BlockSpec((H1, H2), lambda i: (0, 0)),
          pl.BlockSpec((1, H2), lambda i: (0, 0)),
          pl.BlockSpec((H2, 128), lambda i: (0, 0)),
      ],
      out_specs=pl.BlockSpec((BM, 128), lambda i: (i, 0)),
      out_shape=jax.ShapeDtypeStruct((t.shape[0], 128), F32),
  )(t, d, y, w1a, w1b, w1c, b1_2d, w2, b2_2d, w3_2d)


def kernel(title, description, published_year, other_features, emb_table,
           W1, b1, W2, b2, W3, b3):
  del other_features
  table_p = _pad_table(emb_table)
  tidx = title.astype(jnp.int32)
  didx = description.astype(jnp.int32)
  yidx = published_year.astype(jnp.int32)

  pad = EMBP - EMB
  w1a = jnp.pad(W1[:EMB] * (1.0 / T_LEN), ((0, pad), (0, 0)))
  w1b = jnp.pad(W1[EMB:2 * EMB] * (1.0 / D_LEN), ((0, pad), (0, 0)))
  w1c = jnp.pad(W1[2 * EMB:], ((0, pad), (0, 0)))
  b1_2d = b1.reshape(1, H1)
  b2_2d = b2.reshape(1, H2)
  w3p = jnp.pad(W3, ((0, 0), (0, 127)))

  nb = B // 2
  sums = []
  for c in range(2):
    sl = slice(c * nb, (c + 1) * nb)
    sums.append(_sc_bag_sums(table_p, tidx[sl].reshape(-1),
                             didx[sl].reshape(-1), yidx[sl].reshape(-1), nb))
  bf = jnp.bfloat16
  w1a, w1b, w1c, w2b = (w.astype(bf) for w in (w1a, w1b, w1c, W2))
  outs = [_mlp(t_sum, d_sum, y_row, w1a, w1b, w1c, b1_2d, w2b, b2_2d, w3p)
          for t_sum, d_sum, y_row in sums]
  return jnp.concatenate(outs, axis=0)[:, 0] + b3[0]


# speedup vs baseline: 1.0082x; 1.0082x over previous
import functools

import jax
import jax.numpy as jnp
from jax import lax
from jax.experimental import pallas as pl
from jax.experimental.pallas import tpu as pltpu
from jax.experimental.pallas import tpu_sc as plsc

VOCAB = 100000
EMB = 100
EMBP = 128
NVREG = EMBP // 16
B = 16384
NC, NS = 2, 16
NW = NC * NS
BPW = B // NW
T_LEN, D_LEN = 20, 200
DG = 16
TG = 8
H1, H2 = 512, 256
BM = 512
F32 = jnp.float32


def _sc_bag_sums(table_p, tflat, dflat, yflat, nb):
  mesh = plsc.VectorSubcoreMesh(core_axis_name="c", subcore_axis_name="s")
  bpw = nb // NW

  @functools.partial(
      pl.kernel,
      out_type=(
          jax.ShapeDtypeStruct((nb, EMBP), F32),
          jax.ShapeDtypeStruct((nb, EMBP), F32),
          jax.ShapeDtypeStruct((nb, EMBP), F32),
      ),
      mesh=mesh,
      scratch_types=[
          pltpu.VMEM((D_LEN, EMBP), F32),
          pltpu.VMEM((D_LEN, EMBP), F32),
          pltpu.VMEM((D_LEN, EMBP), F32),
          pltpu.VMEM((D_LEN, EMBP), F32),
          pltpu.VMEM((bpw * T_LEN,), jnp.int32),
          pltpu.VMEM((DG * D_LEN,), jnp.int32),
          pltpu.VMEM((DG * D_LEN,), jnp.int32),
          pltpu.VMEM((bpw,), jnp.int32),
          pltpu.VMEM((DG, EMBP), F32),
          pltpu.SemaphoreType.DMA,
          pltpu.SemaphoreType.DMA,
          pltpu.SemaphoreType.DMA,
          pltpu.SemaphoreType.DMA,
          pltpu.SemaphoreType.DMA,
          pltpu.SemaphoreType.DMA,
      ],
  )
  def body(table, tidx_h, didx_h, yidx_h, t_out, d_out, y_out,
           buf0, buf1, buf2, buf3, tidx_v, didx0, didx1, yidx_v, stage,
           semA, semB, semC, semD, semI0, semI1):
    wid = lax.axis_index("s") * NC + lax.axis_index("c")
    bag0 = wid * bpw
    bufs = (buf0, buf1, buf2, buf3)
    sems = (semA, semB, semC, semD)
    didxs = (didx0, didx1)
    semis = (semI0, semI1)

    zeros = tuple(jnp.zeros((16,), F32) for _ in range(NVREG))

    def accum_rows(buf, row0, nrows, unroll):
      def row_add(j, accs):
        r = row0 + unroll * j
        for u in range(unroll):
          accs = tuple(accs[k] + buf[r + u, pl.ds(16 * k, 16)]
                       for k in range(NVREG))
        return accs
      return lax.fori_loop(0, nrows // unroll, row_add, zeros)

    def store_stage(i, accs):
      for k in range(NVREG):
        stage[i, pl.ds(16 * k, 16)] = accs[k]

    def issue_rows(idx_ref, off, n0, n1, buf, sem):
      pltpu.async_copy(table.at[idx_ref.at[pl.ds(off, n0)]],
                       buf.at[pl.ds(0, n0)], sem)
      pltpu.async_copy(table.at[idx_ref.at[pl.ds(off + n0, n1)]],
                       buf.at[pl.ds(n0, n1)], sem)

    def wait_rows(n, buf, sem):
      pltpu.make_async_copy(table.at[pl.ds(0, n)],
                            buf.at[pl.ds(0, n)], sem).wait()

    NYC = bpw // 128
    pltpu.sync_copy(yidx_h.at[pl.ds(bag0, bpw)], yidx_v)
    pltpu.async_copy(table.at[yidx_v.at[pl.ds(0, 128)]],
                     bufs[0].at[pl.ds(0, 128)], sems[0])
    for c in range(NYC):
      if c + 1 < NYC:
        off = pl.multiple_of((c + 1) * 128, 8)
        pltpu.async_copy(table.at[yidx_v.at[pl.ds(off, 128)]],
                         bufs[(c + 1) % 2].at[pl.ds(0, 128)],
                         sems[(c + 1) % 2])
      wait_rows(128, bufs[c % 2], sems[c % 2])
      pltpu.sync_copy(bufs[c % 2].at[pl.ds(0, 128)],
                      y_out.at[pl.ds(bag0 + c * 128, 128)])

    pltpu.sync_copy(tidx_h.at[pl.ds(bag0 * T_LEN, bpw * T_LEN)], tidx_v)
    NTG = bpw // TG
    def issue_tgroup(g, p):
      off = pl.multiple_of(jnp.minimum(g, NTG - 1) * (TG * T_LEN), 8)
      issue_rows(tidx_v, off, 104, 56, bufs[p], sems[p])
    for p in range(3):
      issue_tgroup(p, p)
    def title_quad(sb, _):
      for gp in range(4):
        g = 4 * sb + gp
        issue_tgroup(g + 3, (gp + 3) % 4)
        wait_rows(160, bufs[gp], sems[gp])
        def title_bag(i, _):
          accs = accum_rows(bufs[gp], i * T_LEN, T_LEN, 4)
          store_stage(i, accs)
          return 0
        lax.fori_loop(0, TG, title_bag, 0)
        pltpu.sync_copy(stage.at[pl.ds(0, TG)],
                        t_out.at[pl.ds(bag0 + g * TG, TG)])
      return 0
    lax.fori_loop(0, NTG // 4, title_quad, 0)
    for p in range(3):
      wait_rows(160, bufs[p], sems[p])

    NDG = bpw // DG
    def issue_didx(g, gp):
      goff = bag0 * D_LEN + jnp.minimum(g, NDG - 1) * (DG * D_LEN)
      pltpu.async_copy(didx_h.at[pl.ds(pl.multiple_of(goff, 8), DG * D_LEN)],
                       didxs[gp], semis[gp])
    def wait_didx(gp):
      pltpu.make_async_copy(didx_h.at[pl.ds(0, DG * D_LEN)],
                            didxs[gp], semis[gp]).wait()

    def issue_bag(idx_ref, i, p):
      off = pl.multiple_of(i * D_LEN, 8)
      pltpu.async_copy(table.at[idx_ref.at[pl.ds(off, D_LEN)]],
                       bufs[p], sems[p])

    def do_bag(i, p):
      wait_rows(D_LEN, bufs[p], sems[p])
      store_stage(i, accum_rows(bufs[p], 0, D_LEN, 4))

    issue_didx(0, 0)
    wait_didx(0)
    issue_didx(1, 1)
    for p in range(3):
      issue_bag(didxs[0], p, p)

    def desc_pair(sb, _):
      for gp in range(2):
        g = 2 * sb + gp
        cidx, nidx = didxs[gp], didxs[1 - gp]
        def desc_quad(q, _):
          for p in range(4):
            n = 4 * q + p
            issue_bag(cidx, n + 3, (p + 3) % 4)
            do_bag(n, p)
          return 0
        lax.fori_loop(0, 3, desc_quad, 0)
        issue_bag(cidx, 15, 3)
        do_bag(12, 0)
        wait_didx(1 - gp)
        issue_bag(nidx, 0, 0)
        do_bag(13, 1)
        issue_bag(nidx, 1, 1)
        do_bag(14, 2)
        issue_bag(nidx, 2, 2)
        do_bag(15, 3)
        issue_didx(g + 2, gp)
        pltpu.sync_copy(stage, d_out.at[pl.ds(bag0 + g * DG, DG)])
      return 0
    lax.fori_loop(0, NDG // 2, desc_pair, 0)
    for p in range(3):
      wait_rows(D_LEN, bufs[p], sems[p])
    wait_didx(1)

  return body(table_p, tflat, dflat, yflat)


def _pad_body(x_ref, o_ref):
  o_ref[...] = jnp.concatenate(
      [x_ref[...], jnp.zeros((x_ref.shape[0], EMBP - EMB), F32)], axis=1)


def _pad_table(emb):
  rb = 1000
  return pl.pallas_call(
      _pad_body,
      grid=(VOCAB // rb,),
      in_specs=[pl.BlockSpec((rb, EMB), lambda i: (i, 0))],
      out_specs=pl.BlockSpec((rb, EMBP), lambda i: (i, 0)),
      out_shape=jax.ShapeDtypeStruct((VOCAB, EMBP), F32),
  )(emb)


def _mlp_body(t_ref, d_ref, y_ref, w1a, w1b, w1c, b1, w2, b2, w3, out_ref):
  h = (jnp.dot(t_ref[...], w1a[...], preferred_element_type=F32)
       + jnp.dot(d_ref[...], w1b[...], preferred_element_type=F32)
       + jnp.dot(y_ref[...], w1c[...], preferred_element_type=F32)
       + b1[...])
  h = jnp.maximum(h, 0.0)
  h2 = jnp.maximum(jnp.dot(h, w2[...], preferred_element_type=F32) + b2[...],
                   0.0)
  out_ref[...] = jnp.dot(h2, w3[...], preferred_element_type=F32)


def _mlp(t, d, y, w1a, w1b, w1c, b1_2d, w2, b2_2d, w3_2d):
  grid = t.shape[0] // BM
  return pl.pallas_call(
      _mlp_body,
      grid=(grid,),
      in_specs=[
          pl.BlockSpec((BM, EMBP), lambda i: (i, 0)),
          pl.BlockSpec((BM, EMBP), lambda i: (i, 0)),
          pl.BlockSpec((BM, EMBP), lambda i: (i, 0)),
          pl.BlockSpec((EMBP, H1), lambda i: (0, 0)),
          pl.BlockSpec((EMBP, H1), lambda i: (0, 0)),
          pl.BlockSpec((EMBP, H1), lambda i: (0, 0)),
          pl.BlockSpec((1, H1), lambda i: (0, 0)),
          pl.BlockSpec((H1, H2), lambda i: (0, 0)),
          pl.BlockSpec((1, H2), lambda i: (0, 0)),
          pl.BlockSpec((H2, 128), lambda i: (0, 0)),
      ],
      out_specs=pl.BlockSpec((BM, 128), lambda i: (i, 0)),
      out_shape=jax.ShapeDtypeStruct((t.shape[0], 128), F32),
  )(t, d, y, w1a, w1b, w1c, b1_2d, w2, b2_2d, w3_2d)


def kernel(title, description, published_year, other_features, emb_table,
           W1, b1, W2, b2, W3, b3):
  del other_features
  table_p = _pad_table(emb_table)
  tidx = title.astype(jnp.int32)
  didx = description.astype(jnp.int32)
  yidx = published_year.astype(jnp.int32)

  pad = EMBP - EMB
  w1a = jnp.pad(W1[:EMB] * (1.0 / T_LEN), ((0, pad), (0, 0)))
  w1b = jnp.pad(W1[EMB:2 * EMB] * (1.0 / D_LEN), ((0, pad), (0, 0)))
  w1c = jnp.pad(W1[2 * EMB:], ((0, pad), (0, 0)))
  b1_2d = b1.reshape(1, H1)
  b2_2d = b2.reshape(1, H2)
  w3p = jnp.pad(W3, ((0, 0), (0, 127)))

  nb = B // 2
  sums = []
  for c in range(2):
    sl = slice(c * nb, (c + 1) * nb)
    sums.append(_sc_bag_sums(table_p, tidx[sl].reshape(-1),
                             didx[sl].reshape(-1), yidx[sl].reshape(-1), nb))
  outs = [_mlp(t_sum, d_sum, y_row, w1a, w1b, w1c, b1_2d, W2, b2_2d, w3p)
          for t_sum, d_sum, y_row in sums]
  return jnp.concatenate(outs, axis=0)[:, 0] + b3[0]
